# baseline (speedup 1.0000x reference)
import functools

import numpy as np
import jax
import jax.numpy as jnp
from jax import lax
from jax.experimental import pallas as pl
from jax.experimental.pallas import tpu as pltpu
from jax.experimental.pallas import tpu_sc as plsc

_VOCAB = 1000000
_EMBED = 64
_SEQ = 200
_NSEQ = 4096
_NC, _NS = 2, 16
_NW = _NC * _NS
_C = _NSEQ // _NW
_SCALE = 8.0
_ACH = 128
_NCH = _VOCAB // _ACH
_TAIL = _VOCAB - _NCH * _ACH


def _pe_table(length, depth):
    half = depth / 2
    positions = np.arange(length)[:, np.newaxis]
    depths = np.arange(half)[np.newaxis, :] / half
    angle_rates = 1.0 / (10000.0 ** depths)
    angle_rads = positions * angle_rates
    return np.concatenate(
        [np.sin(angle_rads), np.cos(angle_rads)], axis=-1
    ).astype(np.float32)


_PE_NP = _pe_table(_SEQ, _EMBED).reshape(100, 128)

_MESH = plsc.VectorSubcoreMesh(core_axis_name="c", subcore_axis_name="s")
_PARAMS = pltpu.CompilerParams(needs_layout_passes=False)


@functools.partial(
    pl.kernel,
    mesh=_MESH,
    out_type=jax.ShapeDtypeStruct((_VOCAB, 128), jnp.float32),
    compiler_params=_PARAMS,
    scratch_types=[
        pltpu.VMEM((2, _EMBED, _ACH), jnp.float32),
        pltpu.VMEM((2, _ACH, 128), jnp.float32),
        pltpu.VMEM((32, 128), jnp.float32),
        pltpu.SemaphoreType.DMA,
        pltpu.SemaphoreType.DMA,
        pltpu.SemaphoreType.DMA,
        pltpu.SemaphoreType.DMA,
    ],
)
def _pack_kernel(tt_hbm, tail_hbm, out_hbm, s_v, d_v, tail_v,
                 isem0, isem1, osem0, osem1):
    wid = lax.axis_index("s") * _NC + lax.axis_index("c")
    iota = lax.iota(jnp.int32, 16)
    nk = _NCH // _NW + 1

    isems = (isem0, isem1)
    osems = (osem0, osem1)

    def chunk_of(k):
        return wid + k * jnp.int32(_NW)

    def start_read(k, slot):
        r0 = chunk_of(k) * jnp.int32(_ACH)
        pltpu.async_copy(tt_hbm.at[:, pl.ds(r0, _ACH)], s_v.at[slot],
                         isems[slot])

    def transpose_chunk(k, slot):
        @plsc.parallel_loop(0, _EMBED, unroll=4)
        def _(e):
            ce0 = jnp.full((16,), e, dtype=jnp.int32)
            ce1 = jnp.full((16,), e + jnp.int32(_EMBED), dtype=jnp.int32)
            for m in range(_ACH // 16):
                v = s_v[slot, e, pl.ds(m * 16, 16)]
                iv = iota + jnp.int32(m * 16)
                plsc.store_scatter(d_v.at[slot], [iv, ce0], v)
                plsc.store_scatter(d_v.at[slot], [iv, ce1], v)

    def write_chunk(k, slot):
        r0 = chunk_of(k) * jnp.int32(_ACH)
        pltpu.async_copy(d_v.at[slot], out_hbm.at[pl.ds(r0, _ACH)],
                         osems[slot])

    def wait_read(slot):
        pltpu.make_async_copy(tt_hbm.at[:, pl.ds(0, _ACH)], s_v.at[slot],
                              isems[slot]).wait()

    def wait_write(slot):
        pltpu.make_async_copy(d_v.at[slot], out_hbm.at[pl.ds(0, _ACH)],
                              osems[slot]).wait()

    have0 = chunk_of(0) < _NCH

    @pl.when(have0)
    def _():
        start_read(0, 0)

    def body(q, c):
        k0 = 2 * q
        k1 = k0 + 1

        @pl.when(chunk_of(k1) < _NCH)
        def _():
            start_read(k1, 1)

        @pl.when(chunk_of(k0) < _NCH)
        def _():
            wait_read(0)

            @pl.when(k0 >= 2)
            def _():
                wait_write(0)

            transpose_chunk(k0, 0)
            write_chunk(k0, 0)

        @pl.when(chunk_of(k0 + 2) < _NCH)
        def _():
            start_read(k0 + 2, 0)

        @pl.when(chunk_of(k1) < _NCH)
        def _():
            wait_read(1)

            @pl.when(k1 >= 2)
            def _():
                wait_write(1)

            transpose_chunk(k1, 1)
            write_chunk(k1, 1)

        return c

    nq = (nk + 1) // 2
    lax.fori_loop(0, nq, body, 0)

    wait_write(0)
    wait_write(1)

    @pl.when(wid == 0)
    def _():
        pltpu.sync_copy(tail_hbm, tail_v)

        @plsc.parallel_loop(0, _EMBED, unroll=4)
        def _(e):
            ce0 = jnp.full((16,), e, dtype=jnp.int32)
            ce1 = jnp.full((16,), e + jnp.int32(_EMBED), dtype=jnp.int32)
            for m in range(_TAIL // 16):
                flat = e * jnp.int32(_TAIL) + jnp.int32(16 * m)
                r = lax.shift_right_logical(flat, jnp.int32(7))
                col = lax.bitwise_and(flat, jnp.int32(127))
                v = tail_v[r, pl.ds(col, 16)]
                iv = iota + jnp.int32(m * 16)
                plsc.store_scatter(d_v.at[0], [iv, ce0], v)
                plsc.store_scatter(d_v.at[0], [iv, ce1], v)

        pltpu.sync_copy(d_v.at[0, pl.ds(0, _TAIL)],
                        out_hbm.at[pl.ds(_NCH * _ACH, _TAIL)])


@functools.partial(
    pl.kernel,
    mesh=_MESH,
    out_type=jax.ShapeDtypeStruct((_SEQ, _EMBED, _NSEQ), jnp.float32),
    compiler_params=_PARAMS,
    scratch_types=[
        pltpu.VMEM((_SEQ, _C), jnp.int32),
        pltpu.VMEM((100, 128), jnp.float32),
        pltpu.VMEM((2, _C, 128), jnp.float32),
        pltpu.VMEM((2, _EMBED, _C), jnp.float32),
        pltpu.SemaphoreType.DMA,
        pltpu.SemaphoreType.DMA,
        pltpu.SemaphoreType.DMA,
        pltpu.SemaphoreType.DMA,
    ],
)
def _emb_kernel(xt_hbm, dup_hbm, pe_hbm, out_hbm,
                x_v, pe_v, g_v, t_v, gsem0, gsem1, osem0, osem1):
    wid = lax.axis_index("s") * _NC + lax.axis_index("c")
    s0 = wid * _C
    pltpu.sync_copy(xt_hbm.at[:, pl.ds(s0, _C)], x_v)
    pltpu.sync_copy(pe_hbm, pe_v)

    iota = lax.iota(jnp.int32, 16)

    def start_gather(p, slot, sem):
        return pltpu.async_copy(dup_hbm.at[x_v.at[p]], g_v.at[slot], sem)

    def wait_gather(slot, sem):
        pltpu.make_async_copy(dup_hbm.at[x_v.at[0]], g_v.at[slot],
                              sem).wait()

    def compute_and_write(p, slot, osem, need_owait):
        @pl.when(need_owait)
        def _():
            pltpu.make_async_copy(
                t_v.at[slot], out_hbm.at[p, :, pl.ds(s0, _C)], osem).wait()

        pe4 = []
        for m in range(_EMBED // 16):
            flat = p * jnp.int32(_EMBED) + jnp.int32(16 * m)
            r = lax.shift_right_logical(flat, jnp.int32(7))
            col = lax.bitwise_and(flat, jnp.int32(127))
            pe4.append(pe_v[r, pl.ds(col, 16)])

        ivs = [iota + jnp.int32(16 * m) for m in range(_EMBED // 16)]

        @plsc.parallel_loop(0, _C, unroll=4)
        def _(s):
            sv = jnp.full((16,), s, dtype=jnp.int32)
            for m in range(_EMBED // 16):
                v = g_v[slot, s, pl.ds(m * 16, 16)]
                plsc.store_scatter(t_v.at[slot], [ivs[m], sv],
                                   v * _SCALE + pe4[m])

        pltpu.async_copy(t_v.at[slot], out_hbm.at[p, :, pl.ds(s0, _C)], osem)

    start_gather(0, 0, gsem0)

    def pair_body(q, c):
        p0 = 2 * q
        p1 = p0 + 1

        start_gather(p1, 1, gsem1)
        wait_gather(0, gsem0)
        compute_and_write(p0, 0, osem0, q > 0)

        @pl.when(q + 1 < _SEQ // 2)
        def _():
            start_gather(p0 + 2, 0, gsem0)

        wait_gather(1, gsem1)
        compute_and_write(p1, 1, osem1, q > 0)
        return c

    lax.fori_loop(0, _SEQ // 2, pair_body, 0)

    pltpu.make_async_copy(t_v.at[0],
                          out_hbm.at[_SEQ - 2, :, pl.ds(s0, _C)],
                          osem0).wait()
    pltpu.make_async_copy(t_v.at[1],
                          out_hbm.at[_SEQ - 1, :, pl.ds(s0, _C)],
                          osem1).wait()


def kernel(x, table):
    xt = x.T.astype(jnp.int32)
    tt = table.T
    tail = tt[:, _NCH * _ACH:].reshape(32, 128)
    pe = jnp.asarray(_PE_NP)
    dup = _pack_kernel(tt, tail)
    out_t = _emb_kernel(xt, dup, pe)
    return out_t.transpose(2, 0, 1)


# speedup vs baseline: 1.0054x; 1.0054x over previous
import functools

import numpy as np
import jax
import jax.numpy as jnp
from jax import lax
from jax.experimental import pallas as pl
from jax.experimental.pallas import tpu as pltpu
from jax.experimental.pallas import tpu_sc as plsc

_VOCAB = 1000000
_EMBED = 64
_SEQ = 200
_NSEQ = 4096
_NC, _NS = 2, 16
_NW = _NC * _NS
_C = _NSEQ // _NW
_SCALE = 8.0
_ACH = 128
_NCH = _VOCAB // _ACH
_TAIL = _VOCAB - _NCH * _ACH


def _pe_table(length, depth):
    half = depth / 2
    positions = np.arange(length)[:, np.newaxis]
    depths = np.arange(half)[np.newaxis, :] / half
    angle_rates = 1.0 / (10000.0 ** depths)
    angle_rads = positions * angle_rates
    return np.concatenate(
        [np.sin(angle_rads), np.cos(angle_rads)], axis=-1
    ).astype(np.float32)


_PE_NP = _pe_table(_SEQ, _EMBED).reshape(100, 128)

_MESH = plsc.VectorSubcoreMesh(core_axis_name="c", subcore_axis_name="s")
_PARAMS = pltpu.CompilerParams(needs_layout_passes=False)


@functools.partial(
    pl.kernel,
    mesh=_MESH,
    out_type=jax.ShapeDtypeStruct((_VOCAB, 128), jnp.float32),
    compiler_params=_PARAMS,
    scratch_types=[
        pltpu.VMEM((2, _EMBED, _ACH), jnp.float32),
        pltpu.VMEM((2, _ACH, 129), jnp.float32),
        pltpu.VMEM((32, 128), jnp.float32),
        pltpu.SemaphoreType.DMA,
        pltpu.SemaphoreType.DMA,
        pltpu.SemaphoreType.DMA,
        pltpu.SemaphoreType.DMA,
    ],
)
def _pack_kernel(tt_hbm, tail_hbm, out_hbm, s_v, d_v, tail_v,
                 isem0, isem1, osem0, osem1):
    wid = lax.axis_index("s") * _NC + lax.axis_index("c")
    iota = lax.iota(jnp.int32, 16)
    nk = _NCH // _NW + 1

    isems = (isem0, isem1)
    osems = (osem0, osem1)

    def chunk_of(k):
        return wid + k * jnp.int32(_NW)

    def start_read(k, slot):
        r0 = chunk_of(k) * jnp.int32(_ACH)
        pltpu.async_copy(tt_hbm.at[:, pl.ds(r0, _ACH)], s_v.at[slot],
                         isems[slot])

    def transpose_chunk(k, slot):
        @plsc.parallel_loop(0, _EMBED, unroll=4)
        def _(e):
            ce0 = jnp.full((16,), e, dtype=jnp.int32)
            ce1 = jnp.full((16,), e + jnp.int32(_EMBED), dtype=jnp.int32)
            for m in range(_ACH // 16):
                v = s_v[slot, e, pl.ds(m * 16, 16)]
                iv = iota + jnp.int32(m * 16)
                plsc.store_scatter(d_v.at[slot], [iv, ce0], v)
                plsc.store_scatter(d_v.at[slot], [iv, ce1], v)

    def write_chunk(k, slot):
        r0 = chunk_of(k) * jnp.int32(_ACH)
        pltpu.async_copy(d_v.at[slot, :, pl.ds(0, 128)],
                         out_hbm.at[pl.ds(r0, _ACH)], osems[slot])

    def wait_read(slot):
        pltpu.make_async_copy(tt_hbm.at[:, pl.ds(0, _ACH)], s_v.at[slot],
                              isems[slot]).wait()

    def wait_write(slot):
        pltpu.make_async_copy(d_v.at[slot, :, pl.ds(0, 128)],
                              out_hbm.at[pl.ds(0, _ACH)], osems[slot]).wait()

    have0 = chunk_of(0) < _NCH

    @pl.when(have0)
    def _():
        start_read(0, 0)

    def body(q, c):
        k0 = 2 * q
        k1 = k0 + 1

        @pl.when(chunk_of(k1) < _NCH)
        def _():
            start_read(k1, 1)

        @pl.when(chunk_of(k0) < _NCH)
        def _():
            wait_read(0)

            @pl.when(k0 >= 2)
            def _():
                wait_write(0)

            transpose_chunk(k0, 0)
            write_chunk(k0, 0)

        @pl.when(chunk_of(k0 + 2) < _NCH)
        def _():
            start_read(k0 + 2, 0)

        @pl.when(chunk_of(k1) < _NCH)
        def _():
            wait_read(1)

            @pl.when(k1 >= 2)
            def _():
                wait_write(1)

            transpose_chunk(k1, 1)
            write_chunk(k1, 1)

        return c

    nq = (nk + 1) // 2
    lax.fori_loop(0, nq, body, 0)

    wait_write(0)
    wait_write(1)

    @pl.when(wid == 0)
    def _():
        pltpu.sync_copy(tail_hbm, tail_v)

        @plsc.parallel_loop(0, _EMBED, unroll=4)
        def _(e):
            ce0 = jnp.full((16,), e, dtype=jnp.int32)
            ce1 = jnp.full((16,), e + jnp.int32(_EMBED), dtype=jnp.int32)
            for m in range(_TAIL // 16):
                flat = e * jnp.int32(_TAIL) + jnp.int32(16 * m)
                r = lax.shift_right_logical(flat, jnp.int32(7))
                col = lax.bitwise_and(flat, jnp.int32(127))
                v = tail_v[r, pl.ds(col, 16)]
                iv = iota + jnp.int32(m * 16)
                plsc.store_scatter(d_v.at[0], [iv, ce0], v)
                plsc.store_scatter(d_v.at[0], [iv, ce1], v)

        pltpu.sync_copy(d_v.at[0, pl.ds(0, _TAIL), pl.ds(0, 128)],
                        out_hbm.at[pl.ds(_NCH * _ACH, _TAIL)])


@functools.partial(
    pl.kernel,
    mesh=_MESH,
    out_type=jax.ShapeDtypeStruct((_SEQ, _EMBED, _NSEQ), jnp.float32),
    compiler_params=_PARAMS,
    scratch_types=[
        pltpu.VMEM((_SEQ, _C), jnp.int32),
        pltpu.VMEM((100, 128), jnp.float32),
        pltpu.VMEM((2, _C, 128), jnp.float32),
        pltpu.VMEM((2, _EMBED, _C + 1), jnp.float32),
        pltpu.SemaphoreType.DMA,
        pltpu.SemaphoreType.DMA,
        pltpu.SemaphoreType.DMA,
        pltpu.SemaphoreType.DMA,
    ],
)
def _emb_kernel(xt_hbm, dup_hbm, pe_hbm, out_hbm,
                x_v, pe_v, g_v, t_v, gsem0, gsem1, osem0, osem1):
    wid = lax.axis_index("s") * _NC + lax.axis_index("c")
    s0 = wid * _C
    pltpu.sync_copy(xt_hbm.at[:, pl.ds(s0, _C)], x_v)
    pltpu.sync_copy(pe_hbm, pe_v)

    iota = lax.iota(jnp.int32, 16)

    def start_gather(p, slot, sem):
        return pltpu.async_copy(dup_hbm.at[x_v.at[p]], g_v.at[slot], sem)

    def wait_gather(slot, sem):
        pltpu.make_async_copy(dup_hbm.at[x_v.at[0]], g_v.at[slot],
                              sem).wait()

    def compute_and_write(p, slot, osem, need_owait):
        @pl.when(need_owait)
        def _():
            pltpu.make_async_copy(
                t_v.at[slot, :, pl.ds(0, _C)],
                out_hbm.at[p, :, pl.ds(s0, _C)], osem).wait()

        pe4 = []
        for m in range(_EMBED // 16):
            flat = p * jnp.int32(_EMBED) + jnp.int32(16 * m)
            r = lax.shift_right_logical(flat, jnp.int32(7))
            col = lax.bitwise_and(flat, jnp.int32(127))
            pe4.append(pe_v[r, pl.ds(col, 16)])

        ivs = [iota + jnp.int32(16 * m) for m in range(_EMBED // 16)]

        @plsc.parallel_loop(0, _C, unroll=4)
        def _(s):
            sv = jnp.full((16,), s, dtype=jnp.int32)
            for m in range(_EMBED // 16):
                v = g_v[slot, s, pl.ds(m * 16, 16)]
                plsc.store_scatter(t_v.at[slot], [ivs[m], sv],
                                   v * _SCALE + pe4[m])

        pltpu.async_copy(t_v.at[slot, :, pl.ds(0, _C)],
                         out_hbm.at[p, :, pl.ds(s0, _C)], osem)

    start_gather(0, 0, gsem0)

    def pair_body(q, c):
        p0 = 2 * q
        p1 = p0 + 1

        start_gather(p1, 1, gsem1)
        wait_gather(0, gsem0)
        compute_and_write(p0, 0, osem0, q > 0)

        @pl.when(q + 1 < _SEQ // 2)
        def _():
            start_gather(p0 + 2, 0, gsem0)

        wait_gather(1, gsem1)
        compute_and_write(p1, 1, osem1, q > 0)
        return c

    lax.fori_loop(0, _SEQ // 2, pair_body, 0)

    pltpu.make_async_copy(t_v.at[0, :, pl.ds(0, _C)],
                          out_hbm.at[_SEQ - 2, :, pl.ds(s0, _C)],
                          osem0).wait()
    pltpu.make_async_copy(t_v.at[1, :, pl.ds(0, _C)],
                          out_hbm.at[_SEQ - 1, :, pl.ds(s0, _C)],
                          osem1).wait()


def kernel(x, table):
    xt = x.T.astype(jnp.int32)
    tt = table.T
    tail = tt[:, _NCH * _ACH:].reshape(32, 128)
    pe = jnp.asarray(_PE_NP)
    dup = _pack_kernel(tt, tail)
    out_t = _emb_kernel(xt, dup, pe)
    return out_t.transpose(2, 0, 1)
